# Initial kernel scaffold; baseline (speedup 1.0000x reference)
import jax
import jax.numpy as jnp
from jax.experimental import pallas as pl


def kernel(inp, target):
    raise NotImplementedError("write your pallas kernel here")


# speedup vs baseline: 7.6372x; 7.6372x over previous
import functools

import jax
import jax.numpy as jnp
from jax import lax
from jax.experimental import pallas as pl
from jax.experimental.pallas import tpu as pltpu


def _loss_body(tgt_ref, x_ref, out_ref, m_ref, s_ref, g_ref):
    r = pl.program_id(0)
    c = pl.program_id(1)
    nc = pl.num_programs(1)
    x = x_ref[...]
    R, C = x.shape
    t = tgt_ref[...]
    bm = jnp.max(x, axis=1, keepdims=True)
    cols = c * C + lax.broadcasted_iota(jnp.int32, (R, C), 1)
    gpart = jnp.sum(jnp.where(cols == t, x, 0.0), axis=1, keepdims=True)

    @pl.when(c == 0)
    def _():
        m_ref[...] = bm
        s_ref[...] = jnp.sum(jnp.exp(x - bm), axis=1, keepdims=True)
        g_ref[...] = gpart

    @pl.when(c > 0)
    def _():
        m_old = m_ref[...]
        m_new = jnp.maximum(m_old, bm)
        s_ref[...] = s_ref[...] * jnp.exp(m_old - m_new) + jnp.sum(
            jnp.exp(x - m_new), axis=1, keepdims=True
        )
        m_ref[...] = m_new
        g_ref[...] += gpart

    @pl.when(c == nc - 1)
    def _():
        lse = jnp.log(s_ref[...]) + m_ref[...]
        part = jnp.sum(
            jnp.where(t != 0, lse - g_ref[...], 0.0), axis=(0, 1), keepdims=True
        )

        @pl.when(r == 0)
        def _():
            out_ref[...] = part

        @pl.when(r > 0)
        def _():
            out_ref[...] += part


@functools.partial(jax.jit, static_argnames=("row_block", "col_block", "interpret"))
def _loss_call(inp, tgt, row_block=256, col_block=6400, interpret=False):
    n, v = inp.shape
    out = pl.pallas_call(
        _loss_body,
        grid=(n // row_block, v // col_block),
        in_specs=[
            pl.BlockSpec((row_block, 1), lambda r, c: (r, 0)),
            pl.BlockSpec((row_block, col_block), lambda r, c: (r, c)),
        ],
        out_specs=pl.BlockSpec((1, 1), lambda r, c: (0, 0)),
        out_shape=jax.ShapeDtypeStruct((1, 1), jnp.float32),
        scratch_shapes=[
            pltpu.VMEM((row_block, 1), jnp.float32),
            pltpu.VMEM((row_block, 1), jnp.float32),
            pltpu.VMEM((row_block, 1), jnp.float32),
        ],
        interpret=interpret,
    )(tgt, inp)
    return out[0, 0]


def kernel(inp, target):
    n, v = inp.shape
    tgt = target.astype(jnp.int32).reshape(n, 1)
    return _loss_call(inp, tgt)
